# baseline (speedup 1.0000x reference)
import functools

import jax
import jax.numpy as jnp
from jax import lax
from jax.experimental import pallas as pl
from jax.experimental.pallas import tpu as pltpu
from jax.experimental.pallas import tpu_sc as plsc

N = 4096
C = 256
NV = 3
BQ = 128
NB = N // BQ
CK = 128
NCHK = N // CK
B_OUT = 4 * N

NC = 2
NS = 16
NW = NC * NS
BPW = B_OUT // NW
CH = 128
NCH = BPW // CH


def _sqrt_pos(s):
    return s * lax.rsqrt(jnp.maximum(s, jnp.float32(1.1754944e-38)))


def _argmin_body(params_ref, qx_ref, qy_ref, cx_ref, cy_ref, out_ref):
    sp0 = params_ref[0]
    sp1 = params_ref[1]
    s0 = params_ref[2]
    s1 = params_ref[3]
    qx0 = (qx_ref[...] + sp0) - s0
    qxp = qx_ref[...] - s0
    qy1 = (qy_ref[...] + sp1) - s1
    qyp = qy_ref[...] - s1
    lane = lax.broadcasted_iota(jnp.int32, (BQ, CK), 1)

    inf = jnp.full((BQ, CK), jnp.inf, jnp.float32)
    zero = jnp.zeros((BQ, CK), jnp.int32)
    accv = [inf, inf, inf]
    acci = [zero, zero, zero]
    for c in range(NCHK):
        cxc = cx_ref[c:c + 1, :]
        cyc = cy_ref[c:c + 1, :]
        a = qx0 - cxc
        b = qy1 - cyc
        cc = qxp - cxc
        d = qyp - cyc
        a2 = a * a
        b2 = b * b
        dists = (
            _sqrt_pos(a2 + b2),
            _sqrt_pos(cc * cc + b2),
            _sqrt_pos(a2 + d * d),
        )
        cid = jnp.full((BQ, CK), c, jnp.int32)
        for v in range(NV):
            upd = dists[v] < accv[v]
            accv[v] = jnp.where(upd, dists[v], accv[v])
            acci[v] = jnp.where(upd, cid, acci[v])
    for v in range(NV):
        m = jnp.min(accv[v], axis=1, keepdims=True)
        j = acci[v] * CK + lane
        idx = jnp.min(jnp.where(accv[v] == m, j, N), axis=1)
        out_ref[v, 0, 0, :] = idx


_argmin_call = pl.pallas_call(
    _argmin_body,
    grid=(NB,),
    in_specs=[
        pl.BlockSpec(memory_space=pltpu.SMEM),
        pl.BlockSpec((BQ, 1), lambda b: (b, 0)),
        pl.BlockSpec((BQ, 1), lambda b: (b, 0)),
        pl.BlockSpec((NCHK, CK), lambda b: (0, 0)),
        pl.BlockSpec((NCHK, CK), lambda b: (0, 0)),
    ],
    out_specs=pl.BlockSpec((NV, 1, 1, BQ), lambda b: (0, b, 0, 0)),
    out_shape=jax.ShapeDtypeStruct((NV, NB, 1, BQ), jnp.int32),
)


NIDW = N // BPW
IR = BPW // CH


@functools.lru_cache(maxsize=1)
def _make_sc_gather():
    mesh = plsc.VectorSubcoreMesh(core_axis_name="c", subcore_axis_name="s")

    @functools.partial(
        pl.kernel,
        mesh=mesh,
        out_type=jax.ShapeDtypeStruct((B_OUT, C), jnp.float32),
        scratch_types=[
            pltpu.VMEM((IR, CH), jnp.int32),
            pltpu.VMEM((CH, C), jnp.float32),
            pltpu.VMEM((CH, C), jnp.float32),
            pltpu.SemaphoreType.DMA,
            pltpu.SemaphoreType.DMA,
        ],
    )
    def _sc_gather(table_hbm, idx_hbm, out_hbm, idx_v, rows_a, rows_b, sem_a,
                   sem_b):
        wid = lax.axis_index("s") * NC + lax.axis_index("c")
        base = wid * BPW

        @pl.when(wid < NIDW)
        def _identity():
            pltpu.sync_copy(table_hbm.at[pl.ds(base, BPW)],
                            out_hbm.at[pl.ds(base, BPW)])

        @pl.when(wid >= NIDW)
        def _gather():
            r0 = wid * IR - N // CH
            pltpu.sync_copy(idx_hbm.at[pl.ds(r0, IR)], idx_v)
            bufs = (rows_a, rows_b)
            sems = (sem_a, sem_b)
            pend = [None] * IR
            pend[0] = pltpu.async_copy(table_hbm.at[idx_v.at[0]], rows_a,
                                       sem_a)
            for k in range(IR):
                if k + 1 < IR:
                    pend[k + 1] = pltpu.async_copy(
                        table_hbm.at[idx_v.at[k + 1]], bufs[(k + 1) % 2],
                        sems[(k + 1) % 2])
                pend[k].wait()
                pltpu.sync_copy(bufs[k % 2],
                                out_hbm.at[pl.ds(base + k * CH, CH)])

    return _sc_gather


def kernel(values, coords, spacing, shift):
    params = jnp.concatenate(
        [spacing.astype(jnp.float32), shift.astype(jnp.float32)]
    )
    qx = coords[:, 0:1]
    qy = coords[:, 1:2]
    cx = coords[:, 0].reshape(NCHK, CK)
    cy = coords[:, 1].reshape(NCHK, CK)
    idx = _argmin_call(params, qx, qy, cx, cy).reshape(NV * N // CH, CH)
    return _make_sc_gather()(values, idx)


# speedup vs baseline: 2.0148x; 2.0148x over previous
import functools

import jax
import jax.numpy as jnp
from jax import lax
from jax.experimental import pallas as pl
from jax.experimental.pallas import tpu as pltpu
from jax.experimental.pallas import tpu_sc as plsc

N = 4096
C = 256
NV = 3
BQ = 128
NB = N // BQ
CK = 128
NCHK = N // CK
B_OUT = 4 * N

NC = 2
NS = 16
NW = NC * NS
BPW = B_OUT // NW
CH = 128
NCH = BPW // CH


def _sqrt_pos(s):
    return s * lax.rsqrt(jnp.maximum(s, jnp.float32(1.1754944e-38)))


def _argmin_body(params_ref, qx_ref, qy_ref, cx_ref, cy_ref, out_ref):
    sp0 = params_ref[0]
    sp1 = params_ref[1]
    s0 = params_ref[2]
    s1 = params_ref[3]
    qx0 = (qx_ref[...] + sp0) - s0
    qxp = qx_ref[...] - s0
    qy1 = (qy_ref[...] + sp1) - s1
    qyp = qy_ref[...] - s1
    lane = lax.broadcasted_iota(jnp.int32, (BQ, CK), 1)

    inf = jnp.full((BQ, CK), jnp.inf, jnp.float32)
    zero = jnp.zeros((BQ, CK), jnp.int32)
    accv = [inf, inf, inf]
    acci = [zero, zero, zero]
    for c in range(NCHK):
        cxc = cx_ref[c:c + 1, :]
        cyc = cy_ref[c:c + 1, :]
        a = qx0 - cxc
        b = qy1 - cyc
        cc = qxp - cxc
        d = qyp - cyc
        a2 = a * a
        b2 = b * b
        dists = (
            _sqrt_pos(a2 + b2),
            _sqrt_pos(cc * cc + b2),
            _sqrt_pos(a2 + d * d),
        )
        cid = jnp.full((BQ, CK), c, jnp.int32)
        for v in range(NV):
            upd = dists[v] < accv[v]
            accv[v] = jnp.where(upd, dists[v], accv[v])
            acci[v] = jnp.where(upd, cid, acci[v])
    for v in range(NV):
        m = jnp.min(accv[v], axis=1, keepdims=True)
        j = acci[v] * CK + lane
        idx = jnp.min(jnp.where(accv[v] == m, j, N), axis=1)
        out_ref[v, 0, 0, :] = idx


_argmin_call = pl.pallas_call(
    _argmin_body,
    grid=(NB,),
    in_specs=[
        pl.BlockSpec(memory_space=pltpu.SMEM),
        pl.BlockSpec((BQ, 1), lambda b: (b, 0)),
        pl.BlockSpec((BQ, 1), lambda b: (b, 0)),
        pl.BlockSpec((NCHK, CK), lambda b: (0, 0)),
        pl.BlockSpec((NCHK, CK), lambda b: (0, 0)),
    ],
    out_specs=pl.BlockSpec((NV, 1, 1, BQ), lambda b: (0, b, 0, 0)),
    out_shape=jax.ShapeDtypeStruct((NV, NB, 1, BQ), jnp.int32),
)


NIDW = N // BPW
IR = BPW // CH


@functools.lru_cache(maxsize=1)
def _make_sc_gather():
    mesh = plsc.VectorSubcoreMesh(core_axis_name="c", subcore_axis_name="s")

    @functools.partial(
        pl.kernel,
        mesh=mesh,
        out_type=jax.ShapeDtypeStruct((B_OUT, C), jnp.float32),
        scratch_types=[
            pltpu.VMEM((IR, CH), jnp.int32),
            pltpu.VMEM((CH, C), jnp.float32),
            pltpu.VMEM((CH, C), jnp.float32),
            pltpu.SemaphoreType.DMA,
            pltpu.SemaphoreType.DMA,
        ],
    )
    def _sc_gather(table_hbm, idx_hbm, out_hbm, idx_v, rows_a, rows_b, sem_a,
                   sem_b):
        wid = lax.axis_index("s") * NC + lax.axis_index("c")
        base = wid * BPW

        bufs = (rows_a, rows_b)
        sems = (sem_a, sem_b)

        @pl.when(wid < NIDW)
        def _identity():
            pend = [None] * IR
            pend[0] = pltpu.async_copy(table_hbm.at[pl.ds(base, CH)], rows_a,
                                       sem_a)
            for k in range(IR):
                if k + 1 < IR:
                    pend[k + 1] = pltpu.async_copy(
                        table_hbm.at[pl.ds(base + (k + 1) * CH, CH)],
                        bufs[(k + 1) % 2], sems[(k + 1) % 2])
                pend[k].wait()
                pltpu.sync_copy(bufs[k % 2],
                                out_hbm.at[pl.ds(base + k * CH, CH)])

        @pl.when(wid >= NIDW)
        def _gather():
            r0 = wid * IR - N // CH
            pltpu.sync_copy(idx_hbm.at[pl.ds(r0, IR)], idx_v)
            pend = [None] * IR
            pend[0] = pltpu.async_copy(table_hbm.at[idx_v.at[0]], rows_a,
                                       sem_a)
            for k in range(IR):
                if k + 1 < IR:
                    pend[k + 1] = pltpu.async_copy(
                        table_hbm.at[idx_v.at[k + 1]], bufs[(k + 1) % 2],
                        sems[(k + 1) % 2])
                pend[k].wait()
                pltpu.sync_copy(bufs[k % 2],
                                out_hbm.at[pl.ds(base + k * CH, CH)])

    return _sc_gather


def kernel(values, coords, spacing, shift):
    params = jnp.concatenate(
        [spacing.astype(jnp.float32), shift.astype(jnp.float32)]
    )
    qx = coords[:, 0:1]
    qy = coords[:, 1:2]
    cx = coords[:, 0].reshape(NCHK, CK)
    cy = coords[:, 1].reshape(NCHK, CK)
    idx = _argmin_call(params, qx, qy, cx, cy).reshape(NV * N // CH, CH)
    return _make_sc_gather()(values, idx)


# speedup vs baseline: 2.0349x; 1.0100x over previous
import functools

import jax
import jax.numpy as jnp
from jax import lax
from jax.experimental import pallas as pl
from jax.experimental.pallas import tpu as pltpu
from jax.experimental.pallas import tpu_sc as plsc

N = 4096
C = 256
NV = 3
BQ = 128
NB = N // BQ
CK = 128
NCHK = N // CK
B_OUT = 4 * N

NC = 2
NS = 16
NW = NC * NS
BPW = B_OUT // NW
CH = 128
NCH = BPW // CH


def _sqrt_pos(s):
    return s * lax.rsqrt(jnp.maximum(s, jnp.float32(1.1754944e-38)))


def _argmin_body(spacing_ref, shift_ref, q_ref, cx_ref, cy_ref, out_ref):
    sp0 = spacing_ref[0]
    sp1 = spacing_ref[1]
    s0 = shift_ref[0]
    s1 = shift_ref[1]
    qx = q_ref[:, 0:1]
    qy = q_ref[:, 1:2]
    qx0 = (qx + sp0) - s0
    qxp = qx - s0
    qy1 = (qy + sp1) - s1
    qyp = qy - s1
    lane = lax.broadcasted_iota(jnp.int32, (BQ, CK), 1)

    inf = jnp.full((BQ, CK), jnp.inf, jnp.float32)
    zero = jnp.zeros((BQ, CK), jnp.int32)
    accv = [inf, inf, inf]
    acci = [zero, zero, zero]
    for c in range(NCHK):
        cxc = cx_ref[c:c + 1, :]
        cyc = cy_ref[c:c + 1, :]
        a = qx0 - cxc
        b = qy1 - cyc
        cc = qxp - cxc
        d = qyp - cyc
        a2 = a * a
        b2 = b * b
        dists = (
            _sqrt_pos(a2 + b2),
            _sqrt_pos(cc * cc + b2),
            _sqrt_pos(a2 + d * d),
        )
        cid = jnp.full((BQ, CK), c, jnp.int32)
        for v in range(NV):
            upd = dists[v] < accv[v]
            accv[v] = jnp.where(upd, dists[v], accv[v])
            acci[v] = jnp.where(upd, cid, acci[v])
    for v in range(NV):
        m = jnp.min(accv[v], axis=1, keepdims=True)
        j = acci[v] * CK + lane
        idx = jnp.min(jnp.where(accv[v] == m, j, N), axis=1)
        out_ref[v, 0, 0, :] = idx


_argmin_call = pl.pallas_call(
    _argmin_body,
    grid=(NB,),
    in_specs=[
        pl.BlockSpec(memory_space=pltpu.SMEM),
        pl.BlockSpec(memory_space=pltpu.SMEM),
        pl.BlockSpec((BQ, 2), lambda b: (b, 0)),
        pl.BlockSpec((NCHK, CK), lambda b: (0, 0)),
        pl.BlockSpec((NCHK, CK), lambda b: (0, 0)),
    ],
    out_specs=pl.BlockSpec((NV, 1, 1, BQ), lambda b: (0, b, 0, 0)),
    out_shape=jax.ShapeDtypeStruct((NV, NB, 1, BQ), jnp.int32),
)


NIDW = N // BPW
IR = BPW // CH


@functools.lru_cache(maxsize=1)
def _make_sc_gather():
    mesh = plsc.VectorSubcoreMesh(core_axis_name="c", subcore_axis_name="s")

    @functools.partial(
        pl.kernel,
        mesh=mesh,
        out_type=jax.ShapeDtypeStruct((B_OUT, C), jnp.float32),
        scratch_types=[
            pltpu.VMEM((IR, CH), jnp.int32),
            pltpu.VMEM((CH, C), jnp.float32),
            pltpu.VMEM((CH, C), jnp.float32),
            pltpu.SemaphoreType.DMA,
            pltpu.SemaphoreType.DMA,
        ],
    )
    def _sc_gather(table_hbm, idx_hbm, out_hbm, idx_v, rows_a, rows_b, sem_a,
                   sem_b):
        wid = lax.axis_index("s") * NC + lax.axis_index("c")
        base = wid * BPW

        bufs = (rows_a, rows_b)
        sems = (sem_a, sem_b)

        @pl.when(wid < NIDW)
        def _identity():
            pend = [None] * IR
            pend[0] = pltpu.async_copy(table_hbm.at[pl.ds(base, CH)], rows_a,
                                       sem_a)
            for k in range(IR):
                if k + 1 < IR:
                    pend[k + 1] = pltpu.async_copy(
                        table_hbm.at[pl.ds(base + (k + 1) * CH, CH)],
                        bufs[(k + 1) % 2], sems[(k + 1) % 2])
                pend[k].wait()
                pltpu.sync_copy(bufs[k % 2],
                                out_hbm.at[pl.ds(base + k * CH, CH)])

        @pl.when(wid >= NIDW)
        def _gather():
            r0 = wid * IR - N // CH
            pltpu.sync_copy(idx_hbm.at[pl.ds(r0, IR)], idx_v)
            pend = [None] * IR
            pend[0] = pltpu.async_copy(table_hbm.at[idx_v.at[0]], rows_a,
                                       sem_a)
            for k in range(IR):
                if k + 1 < IR:
                    pend[k + 1] = pltpu.async_copy(
                        table_hbm.at[idx_v.at[k + 1]], bufs[(k + 1) % 2],
                        sems[(k + 1) % 2])
                pend[k].wait()
                pltpu.sync_copy(bufs[k % 2],
                                out_hbm.at[pl.ds(base + k * CH, CH)])

    return _sc_gather


def kernel(values, coords, spacing, shift):
    cx = coords[:, 0].reshape(NCHK, CK)
    cy = coords[:, 1].reshape(NCHK, CK)
    idx = _argmin_call(spacing, shift, coords, cx, cy)
    return _make_sc_gather()(values, idx.reshape(NV * N // CH, CH))


# speedup vs baseline: 2.0840x; 1.0241x over previous
import functools

import jax
import jax.numpy as jnp
from jax import lax
from jax.experimental import pallas as pl
from jax.experimental.pallas import tpu as pltpu
from jax.experimental.pallas import tpu_sc as plsc

N = 4096
C = 256
NV = 3
BQ = 128
NB = N // BQ
CK = 128
NCHK = N // CK
B_OUT = 4 * N

NC = 2
NS = 16
NW = NC * NS
BPW = B_OUT // NW
CH = 128
NCH = BPW // CH


def _sqrt_pos(s):
    return s * lax.rsqrt(jnp.maximum(s, jnp.float32(1.1754944e-38)))


CS = 8
NCH2 = N // CS


def _argmin_body(spacing_ref, shift_ref, q_ref, cx_ref, cy_ref, out_ref):
    sp0 = spacing_ref[0]
    sp1 = spacing_ref[1]
    s0 = shift_ref[0]
    s1 = shift_ref[1]
    qx = q_ref[0:1, :]
    qy = q_ref[1:2, :]
    qx0 = (qx + sp0) - s0
    qxp = qx - s0
    qy1 = (qy + sp1) - s1
    qyp = qy - s1
    srow = lax.broadcasted_iota(jnp.int32, (CS, BQ), 0)

    inf = jnp.full((CS, BQ), jnp.inf, jnp.float32)
    zero = jnp.zeros((CS, BQ), jnp.int32)
    accv = [inf, inf, inf]
    acci = [zero, zero, zero]
    for c in range(NCH2):
        cxc = cx_ref[c]
        cyc = cy_ref[c]
        a = qx0 - cxc
        b = qy1 - cyc
        cc = qxp - cxc
        d = qyp - cyc
        a2 = a * a
        b2 = b * b
        dists = (
            _sqrt_pos(a2 + b2),
            _sqrt_pos(cc * cc + b2),
            _sqrt_pos(a2 + d * d),
        )
        cid = jnp.full((CS, BQ), c, jnp.int32)
        for v in range(NV):
            upd = dists[v] < accv[v]
            accv[v] = jnp.where(upd, dists[v], accv[v])
            acci[v] = jnp.where(upd, cid, acci[v])
    for v in range(NV):
        m = jnp.min(accv[v], axis=0, keepdims=True)
        j = acci[v] * CS + srow
        idx = jnp.min(jnp.where(accv[v] == m, j, N), axis=0)
        out_ref[v, 0, 0, :] = idx


_argmin_call = pl.pallas_call(
    _argmin_body,
    grid=(NB,),
    in_specs=[
        pl.BlockSpec(memory_space=pltpu.SMEM),
        pl.BlockSpec(memory_space=pltpu.SMEM),
        pl.BlockSpec((2, BQ), lambda b: (0, b)),
        pl.BlockSpec((NCH2, CS, 1), lambda b: (0, 0, 0)),
        pl.BlockSpec((NCH2, CS, 1), lambda b: (0, 0, 0)),
    ],
    out_specs=pl.BlockSpec((NV, 1, 1, BQ), lambda b: (0, b, 0, 0)),
    out_shape=jax.ShapeDtypeStruct((NV, NB, 1, BQ), jnp.int32),
)


NIDW = N // BPW
IR = BPW // CH


@functools.lru_cache(maxsize=1)
def _make_sc_gather():
    mesh = plsc.VectorSubcoreMesh(core_axis_name="c", subcore_axis_name="s")

    @functools.partial(
        pl.kernel,
        mesh=mesh,
        out_type=jax.ShapeDtypeStruct((B_OUT, C), jnp.float32),
        scratch_types=[
            pltpu.VMEM((IR, CH), jnp.int32),
            pltpu.VMEM((CH, C), jnp.float32),
            pltpu.VMEM((CH, C), jnp.float32),
            pltpu.SemaphoreType.DMA,
            pltpu.SemaphoreType.DMA,
        ],
    )
    def _sc_gather(table_hbm, idx_hbm, out_hbm, idx_v, rows_a, rows_b, sem_a,
                   sem_b):
        wid = lax.axis_index("s") * NC + lax.axis_index("c")
        base = wid * BPW

        bufs = (rows_a, rows_b)
        sems = (sem_a, sem_b)

        @pl.when(wid < NIDW)
        def _identity():
            pend = [None] * IR
            pend[0] = pltpu.async_copy(table_hbm.at[pl.ds(base, CH)], rows_a,
                                       sem_a)
            for k in range(IR):
                if k + 1 < IR:
                    pend[k + 1] = pltpu.async_copy(
                        table_hbm.at[pl.ds(base + (k + 1) * CH, CH)],
                        bufs[(k + 1) % 2], sems[(k + 1) % 2])
                pend[k].wait()
                pltpu.sync_copy(bufs[k % 2],
                                out_hbm.at[pl.ds(base + k * CH, CH)])

        @pl.when(wid >= NIDW)
        def _gather():
            r0 = wid * IR - N // CH
            pltpu.sync_copy(idx_hbm.at[pl.ds(r0, IR)], idx_v)
            pend = [None] * IR
            pend[0] = pltpu.async_copy(table_hbm.at[idx_v.at[0]], rows_a,
                                       sem_a)
            for k in range(IR):
                if k + 1 < IR:
                    pend[k + 1] = pltpu.async_copy(
                        table_hbm.at[idx_v.at[k + 1]], bufs[(k + 1) % 2],
                        sems[(k + 1) % 2])
                pend[k].wait()
                pltpu.sync_copy(bufs[k % 2],
                                out_hbm.at[pl.ds(base + k * CH, CH)])

    return _sc_gather


def kernel(values, coords, spacing, shift):
    qt = coords.T
    cx = coords[:, 0].reshape(NCH2, CS, 1)
    cy = coords[:, 1].reshape(NCH2, CS, 1)
    idx = _argmin_call(spacing, shift, qt, cx, cy)
    return _make_sc_gather()(values, idx.reshape(NV * N // CH, CH))
